# baseline (speedup 1.0000x reference)
import jax
import jax.numpy as jnp
from jax import lax
from jax.experimental import pallas as pl
from jax.experimental.pallas import tpu as pltpu
from jax.experimental.pallas import tpu_sc as plsc

B = 4
E = 8
C = 512
HW = 256
T = B * HW
EPAD = 128
BLK = 128
NB = T // BLK + E
PAD = NB * BLK
NC, NS, L = 2, 16, 16
NW = NC * NS
CH = T // NW
NV = T // L


def _gate_body(x_ref, gw1_ref, gb1_ref, gw2_ref, gb2_ref,
               idx_ref, p_ref, tokb_ref):
    xs = x_ref[...]
    tok = jnp.transpose(xs, (0, 2, 1)).reshape(T, C)
    tokb_ref[...] = tok.astype(jnp.bfloat16)
    g1 = jax.nn.gelu(
        jax.lax.dot_general(tok, gw1_ref[...], (((1,), (0,)), ((), ())),
                            preferred_element_type=jnp.float32)
        + gb1_ref[...])
    gw2p = jnp.concatenate(
        [gw2_ref[...], jnp.zeros((C, EPAD - E), jnp.float32)], axis=1)
    logits = jax.lax.dot_general(g1, gw2p, (((1,), (0,)), ((), ())),
                                 preferred_element_type=jnp.float32)
    logits = logits + jnp.concatenate(
        [gb2_ref[...], jnp.zeros((1, EPAD - E), jnp.float32)], axis=1)
    col = jax.lax.broadcasted_iota(jnp.int32, (T, EPAD), 1)
    logits = jnp.where(col < E, logits, -1e30)
    m = jnp.max(logits, axis=1, keepdims=True)
    ex = jnp.exp(logits - m)
    denom = jnp.sum(ex, axis=1, keepdims=True)
    p_ref[...] = 1.0 / denom
    idx_ref[...] = jnp.min(jnp.where(logits == m, col, EPAD),
                           axis=1, keepdims=True)


def _gate(s32r, gate_w1, gate_b1, gate_w2, gate_b2):
    return pl.pallas_call(
        _gate_body,
        out_shape=(jax.ShapeDtypeStruct((T, 1), jnp.int32),
                   jax.ShapeDtypeStruct((T, 1), jnp.float32),
                   jax.ShapeDtypeStruct((T, C), jnp.bfloat16)),
    )(s32r, gate_w1, gate_b1.reshape(1, C), gate_w2, gate_b2.reshape(1, E))


def _vgather(v, i):
    return lax.gather(
        v, i[:, None],
        lax.GatherDimensionNumbers(offset_dims=(), collapsed_slice_dims=(0,),
                                   start_index_map=(0,)),
        slice_sizes=(1,),
        mode=lax.GatherScatterMode.PROMISE_IN_BOUNDS)


def _worker_dispatch_math(read_vreg, wid):
    iota = lax.iota(jnp.int32, L)
    zero = jnp.zeros((L,), jnp.int32)

    def hist_step(k, carry):
        tot, pre = carry
        v = read_vreg(k)
        flag = jnp.where(k < 2 * wid, 1, 0)
        new_tot, new_pre = [], []
        for e in range(E):
            m = jnp.where(v == e, 1, 0)
            new_tot.append(tot[e] + m)
            new_pre.append(pre[e] + m * flag)
        return tuple(new_tot), tuple(new_pre)

    tot, pre = lax.fori_loop(0, NV, hist_step,
                             (tuple(zero for _ in range(E)),
                              tuple(zero for _ in range(E))))
    hv = zero
    pv = zero
    for e in range(E):
        lane = jnp.where(iota == e, 1, 0)
        hv = hv + lane * jnp.sum(tot[e])
        pv = pv + lane * jnp.sum(pre[e])

    nblk = (hv + (BLK - 1)) // BLK
    bs_incl = jnp.cumsum(nblk)
    padded_off = (bs_incl - nblk) * BLK
    base = padded_off + pv

    a = read_vreg(2 * wid)
    b = read_vreg(2 * wid + 1)
    cnt_a = zero
    intra_a = zero
    intra_b = zero
    for e in range(E):
        ma = jnp.where(a == e, 1, 0)
        mb = jnp.where(b == e, 1, 0)
        ca = jnp.cumsum(ma)
        cb = jnp.cumsum(mb)
        intra_a = jnp.where(a == e, ca - 1, intra_a)
        cnt_a = cnt_a + jnp.where(iota == e, 1, 0) * jnp.sum(ma)
        intra_b = jnp.where(b == e, cb - 1, intra_b)
    dest_a = _vgather(base, a) + intra_a
    dest_b = _vgather(base + cnt_a, b) + intra_b

    shifted = _vgather(bs_incl, jnp.maximum(iota - 1, 0))
    bsx = shifted * jnp.where(iota == 0, 0, 1)
    bsx = bsx * jnp.where(iota > E, 0, 1) + jnp.where(iota > E, NB, 0)
    return dest_a, dest_b, bsx, hv


def _dispatch_body(idx_hbm, tok_hbm, xpad_hbm, dest_hbm,
                   bs_hbm, idx_all, dest_v, rows_v, bs_v, sem):
    wid = lax.axis_index("s") * NC + lax.axis_index("c")
    pltpu.sync_copy(idx_hbm, idx_all)
    read = lambda k: idx_all[pl.ds(k * L, L)]
    dest_a, dest_b, bsx, hv = _worker_dispatch_math(read, wid)

    dest_v[pl.ds(0, L)] = dest_a
    dest_v[pl.ds(L, L)] = dest_b
    pltpu.sync_copy(dest_v, dest_hbm.at[pl.ds(wid * CH, CH)])

    pltpu.sync_copy(tok_hbm.at[pl.ds(wid * CH, CH)], rows_v)
    pltpu.async_copy(rows_v, xpad_hbm.at[dest_v], sem).wait()

    @pl.when(wid == 0)
    def _():
        bs_v[pl.ds(0, L)] = bsx
        bs_v[pl.ds(L, L)] = hv
        pltpu.sync_copy(bs_v, bs_hbm)


def _dispatch(idx, tok_bf):
    mesh = plsc.VectorSubcoreMesh(core_axis_name="c", subcore_axis_name="s",
                                  num_cores=NC, num_subcores=NS)
    return pl.kernel(
        _dispatch_body,
        out_type=(jax.ShapeDtypeStruct((PAD, C // 2), jnp.int32),
                  jax.ShapeDtypeStruct((T,), jnp.int32),
                  jax.ShapeDtypeStruct((32,), jnp.int32)),
        mesh=mesh,
        scratch_types=[
            pltpu.VMEM((T,), jnp.int32),
            pltpu.VMEM((CH,), jnp.int32),
            pltpu.VMEM((CH, C // 2), jnp.int32),
            pltpu.VMEM((2 * L,), jnp.int32),
            pltpu.SemaphoreType.DMA,
        ],
        compiler_params=pltpu.CompilerParams(needs_layout_passes=False),
    )(idx, tok_bf)


def _experts_body(bs_ref, x_ref, w1_ref, b1_ref, w2_ref, b2_ref, out_ref):
    j = pl.program_id(0)
    used = bs_ref[E]

    @pl.when(j < used)
    def _():
        e = _expert_of_block(j, bs_ref)
        h = jax.nn.gelu(
            jax.lax.dot_general(x_ref[...], w1_ref[0].astype(jnp.bfloat16),
                                (((1,), (0,)), ((), ())),
                                preferred_element_type=jnp.float32)
            + b1_ref[0])
        y = jax.lax.dot_general(
            h.astype(jnp.bfloat16), w2_ref[0].astype(jnp.bfloat16),
            (((1,), (0,)), ((), ())),
            preferred_element_type=jnp.float32) + b2_ref[0]
        row = jax.lax.broadcasted_iota(jnp.int32, (BLK, 1), 0)
        row_in_region = row + (j - _bs_at(bs_ref, e)) * BLK
        cnt = _cnt_at(bs_ref, e)
        out_ref[...] = jnp.where(row_in_region < cnt, y, 0.0).astype(
            jnp.bfloat16)

    @pl.when(j >= used)
    def _():
        out_ref[...] = jnp.zeros((BLK, C), jnp.bfloat16)


def _expert_of_block(j, bs_ref):
    e = jnp.int32(0)
    for k in range(1, E):
        e = e + jnp.where(j >= bs_ref[k], 1, 0).astype(jnp.int32)
    return e


def _bs_at(bs_ref, e):
    v = jnp.int32(0)
    for k in range(E):
        v = v + jnp.where(e == k, bs_ref[k], 0).astype(jnp.int32)
    return v


def _cnt_at(bs_ref, e):
    v = jnp.int32(0)
    for k in range(E):
        v = v + jnp.where(e == k, bs_ref[L + k], 0).astype(jnp.int32)
    return v


def _grouped_experts(x_padded, bs, exp_w1, exp_b1, exp_w2, exp_b2):
    grid_spec = pltpu.PrefetchScalarGridSpec(
        num_scalar_prefetch=1,
        grid=(NB,),
        in_specs=[
            pl.BlockSpec((BLK, C), lambda j, bs_ref: (j, 0)),
            pl.BlockSpec((1, C, C),
                         lambda j, bs_ref: (_expert_of_block(j, bs_ref), 0, 0)),
            pl.BlockSpec((1, 1, C),
                         lambda j, bs_ref: (_expert_of_block(j, bs_ref), 0, 0)),
            pl.BlockSpec((1, C, C),
                         lambda j, bs_ref: (_expert_of_block(j, bs_ref), 0, 0)),
            pl.BlockSpec((1, 1, C),
                         lambda j, bs_ref: (_expert_of_block(j, bs_ref), 0, 0)),
        ],
        out_specs=pl.BlockSpec((BLK, C), lambda j, bs_ref: (j, 0)),
    )
    return pl.pallas_call(
        _experts_body,
        grid_spec=grid_spec,
        out_shape=jax.ShapeDtypeStruct((PAD, C), jnp.bfloat16),
    )(bs, x_padded, exp_w1, exp_b1.reshape(E, 1, C),
      exp_w2, exp_b2.reshape(E, 1, C))


def _epilogue_body(y_ref, dest_ref, p_ref, s_ref, out_ref):
    dest_row = dest_ref[...]
    jrow = jax.lax.broadcasted_iota(jnp.int32, (PAD, T), 0)
    perm = (jrow == dest_row).astype(jnp.bfloat16)
    y_tok = jax.lax.dot_general(perm, y_ref[...], (((0,), (0,)), ((), ())),
                                preferred_element_type=jnp.float32)
    z = y_tok * p_ref[...]
    z3 = z.reshape(B, HW, C)
    out_ref[...] = jnp.transpose(z3, (0, 2, 1)) + s_ref[...]


def _epilogue(y_padded, dest_row, p, s32r):
    return pl.pallas_call(
        _epilogue_body,
        out_shape=jax.ShapeDtypeStruct((B, C, HW), jnp.float32),
    )(y_padded, dest_row, p, s32r)


def kernel(s4, s8, s16, s32, gate_w1, gate_b1, gate_w2, gate_b2,
           exp_w1, exp_b1, exp_w2, exp_b2):
    s32r = s32.reshape(B, C, HW)

    idx2, p2, tok_bf = _gate(s32r, gate_w1, gate_b1, gate_w2, gate_b2)
    idx_flat = idx2.reshape(T)
    tok_i = lax.bitcast_convert_type(tok_bf.reshape(T, C // 2, 2), jnp.int32)
    xpad_i, dest, bs = _dispatch(idx_flat, tok_i)
    x_padded = lax.bitcast_convert_type(xpad_i, jnp.bfloat16).reshape(PAD, C)
    y_padded = _grouped_experts(x_padded, bs, exp_w1, exp_b1, exp_w2, exp_b2)
    s32_out = _epilogue(y_padded, dest.reshape(1, T), p2,
                        s32r).reshape(B, C, 16, 16)

    return (s4, s8, s16, s32_out)


# speedup vs baseline: 1.2659x; 1.2659x over previous
import jax
import jax.numpy as jnp
from jax import lax
from jax.experimental import pallas as pl
from jax.experimental.pallas import tpu as pltpu
from jax.experimental.pallas import tpu_sc as plsc

B = 4
E = 8
C = 512
HW = 256
T = B * HW
EPAD = 128
BLK = 128
NB = T // BLK + E
PAD = NB * BLK
NC, NS, L = 2, 16, 16
NW = NC * NS
CH = T // NW
NV = T // L


def _gate_body(x_ref, gw1_ref, gb1_ref, gw2_ref, gb2_ref,
               idx_ref, p_ref, tok_ref):
    xs = x_ref[...]
    tok = jnp.transpose(xs, (0, 2, 1)).reshape(T, C)
    tok_ref[...] = tok
    g1 = jax.nn.gelu(
        jax.lax.dot_general(tok, gw1_ref[...], (((1,), (0,)), ((), ())),
                            preferred_element_type=jnp.float32)
        + gb1_ref[...])
    gw2p = jnp.concatenate(
        [gw2_ref[...], jnp.zeros((C, EPAD - E), jnp.float32)], axis=1)
    logits = jax.lax.dot_general(g1, gw2p, (((1,), (0,)), ((), ())),
                                 preferred_element_type=jnp.float32)
    logits = logits + jnp.concatenate(
        [gb2_ref[...], jnp.zeros((1, EPAD - E), jnp.float32)], axis=1)
    col = jax.lax.broadcasted_iota(jnp.int32, (T, EPAD), 1)
    logits = jnp.where(col < E, logits, -1e30)
    m = jnp.max(logits, axis=1, keepdims=True)
    ex = jnp.exp(logits - m)
    denom = jnp.sum(ex, axis=1, keepdims=True)
    p_ref[...] = 1.0 / denom
    idx_ref[...] = jnp.min(jnp.where(logits == m, col, EPAD),
                           axis=1, keepdims=True)


def _gate(s32r, gate_w1, gate_b1, gate_w2, gate_b2):
    return pl.pallas_call(
        _gate_body,
        out_shape=(jax.ShapeDtypeStruct((T, 1), jnp.int32),
                   jax.ShapeDtypeStruct((T, 1), jnp.float32),
                   jax.ShapeDtypeStruct((T, C), jnp.float32)),
    )(s32r, gate_w1, gate_b1.reshape(1, C), gate_w2, gate_b2.reshape(1, E))


def _vgather(v, i):
    return lax.gather(
        v, i[:, None],
        lax.GatherDimensionNumbers(offset_dims=(), collapsed_slice_dims=(0,),
                                   start_index_map=(0,)),
        slice_sizes=(1,),
        mode=lax.GatherScatterMode.PROMISE_IN_BOUNDS)


def _worker_dispatch_math(read_vreg, wid):
    iota = lax.iota(jnp.int32, L)
    zero = jnp.zeros((L,), jnp.int32)

    def hist_step(k, carry):
        tot, pre = carry
        v = read_vreg(k)
        flag = jnp.where(k < 2 * wid, 1, 0)
        new_tot, new_pre = [], []
        for e in range(E):
            m = jnp.where(v == e, 1, 0)
            new_tot.append(tot[e] + m)
            new_pre.append(pre[e] + m * flag)
        return tuple(new_tot), tuple(new_pre)

    tot, pre = lax.fori_loop(0, NV, hist_step,
                             (tuple(zero for _ in range(E)),
                              tuple(zero for _ in range(E))))
    hv = zero
    pv = zero
    for e in range(E):
        lane = jnp.where(iota == e, 1, 0)
        hv = hv + lane * jnp.sum(tot[e])
        pv = pv + lane * jnp.sum(pre[e])

    nblk = (hv + (BLK - 1)) // BLK
    bs_incl = jnp.cumsum(nblk)
    padded_off = (bs_incl - nblk) * BLK
    base = padded_off + pv

    a = read_vreg(2 * wid)
    b = read_vreg(2 * wid + 1)
    cnt_a = zero
    intra_a = zero
    intra_b = zero
    for e in range(E):
        ma = jnp.where(a == e, 1, 0)
        mb = jnp.where(b == e, 1, 0)
        ca = jnp.cumsum(ma)
        cb = jnp.cumsum(mb)
        intra_a = jnp.where(a == e, ca - 1, intra_a)
        cnt_a = cnt_a + jnp.where(iota == e, 1, 0) * jnp.sum(ma)
        intra_b = jnp.where(b == e, cb - 1, intra_b)
    dest_a = _vgather(base, a) + intra_a
    dest_b = _vgather(base + cnt_a, b) + intra_b

    shifted = _vgather(bs_incl, jnp.maximum(iota - 1, 0))
    bsx = shifted * jnp.where(iota == 0, 0, 1)
    bsx = bsx * jnp.where(iota > E, 0, 1) + jnp.where(iota > E, NB, 0)
    return dest_a, dest_b, bsx, hv


def _dispatch_body(idx_hbm, tok_hbm, xpad_hbm, dest_hbm,
                   bs_hbm, idx_all, dest_v, rows_v, bs_v, sem):
    wid = lax.axis_index("s") * NC + lax.axis_index("c")
    pltpu.sync_copy(idx_hbm, idx_all)
    read = lambda k: idx_all[pl.ds(k * L, L)]
    dest_a, dest_b, bsx, hv = _worker_dispatch_math(read, wid)

    dest_v[pl.ds(0, L)] = dest_a
    dest_v[pl.ds(L, L)] = dest_b
    pltpu.sync_copy(dest_v, dest_hbm.at[pl.ds(wid * CH, CH)])

    pltpu.sync_copy(tok_hbm.at[pl.ds(wid * CH, CH)], rows_v)
    pltpu.async_copy(rows_v, xpad_hbm.at[dest_v], sem).wait()

    @pl.when(wid == 0)
    def _():
        bs_v[pl.ds(0, L)] = bsx
        bs_v[pl.ds(L, L)] = hv
        pltpu.sync_copy(bs_v, bs_hbm)


def _dispatch(idx, tok_bf):
    mesh = plsc.VectorSubcoreMesh(core_axis_name="c", subcore_axis_name="s",
                                  num_cores=NC, num_subcores=NS)
    return pl.kernel(
        _dispatch_body,
        out_type=(jax.ShapeDtypeStruct((PAD, C), jnp.float32),
                  jax.ShapeDtypeStruct((T,), jnp.int32),
                  jax.ShapeDtypeStruct((32,), jnp.int32)),
        mesh=mesh,
        scratch_types=[
            pltpu.VMEM((T,), jnp.int32),
            pltpu.VMEM((CH,), jnp.int32),
            pltpu.VMEM((CH, C), jnp.float32),
            pltpu.VMEM((2 * L,), jnp.int32),
            pltpu.SemaphoreType.DMA,
        ],
        compiler_params=pltpu.CompilerParams(needs_layout_passes=False),
    )(idx, tok_bf)


def _experts_body(bs_ref, x_ref, w1_ref, b1_ref, w2_ref, b2_ref, out_ref):
    j = pl.program_id(0)
    used = bs_ref[E]

    @pl.when(j < used)
    def _():
        e = _expert_of_block(j, bs_ref)
        h = jax.nn.gelu(
            jax.lax.dot_general(x_ref[...].astype(jnp.bfloat16),
                                w1_ref[0].astype(jnp.bfloat16),
                                (((1,), (0,)), ((), ())),
                                preferred_element_type=jnp.float32)
            + b1_ref[0])
        y = jax.lax.dot_general(
            h.astype(jnp.bfloat16), w2_ref[0].astype(jnp.bfloat16),
            (((1,), (0,)), ((), ())),
            preferred_element_type=jnp.float32) + b2_ref[0]
        row = jax.lax.broadcasted_iota(jnp.int32, (BLK, 1), 0)
        row_in_region = row + (j - _bs_at(bs_ref, e)) * BLK
        cnt = _cnt_at(bs_ref, e)
        out_ref[...] = jnp.where(row_in_region < cnt, y, 0.0).astype(
            jnp.bfloat16)

    @pl.when(j >= used)
    def _():
        out_ref[...] = jnp.zeros((BLK, C), jnp.bfloat16)


def _expert_of_block(j, bs_ref):
    e = jnp.int32(0)
    for k in range(1, E):
        e = e + jnp.where(j >= bs_ref[k], 1, 0).astype(jnp.int32)
    return e


def _bs_at(bs_ref, e):
    v = jnp.int32(0)
    for k in range(E):
        v = v + jnp.where(e == k, bs_ref[k], 0).astype(jnp.int32)
    return v


def _cnt_at(bs_ref, e):
    v = jnp.int32(0)
    for k in range(E):
        v = v + jnp.where(e == k, bs_ref[L + k], 0).astype(jnp.int32)
    return v


def _grouped_experts(x_padded, bs, exp_w1, exp_b1, exp_w2, exp_b2):
    grid_spec = pltpu.PrefetchScalarGridSpec(
        num_scalar_prefetch=1,
        grid=(NB,),
        in_specs=[
            pl.BlockSpec((BLK, C), lambda j, bs_ref: (j, 0)),
            pl.BlockSpec((1, C, C),
                         lambda j, bs_ref: (_expert_of_block(j, bs_ref), 0, 0)),
            pl.BlockSpec((1, 1, C),
                         lambda j, bs_ref: (_expert_of_block(j, bs_ref), 0, 0)),
            pl.BlockSpec((1, C, C),
                         lambda j, bs_ref: (_expert_of_block(j, bs_ref), 0, 0)),
            pl.BlockSpec((1, 1, C),
                         lambda j, bs_ref: (_expert_of_block(j, bs_ref), 0, 0)),
        ],
        out_specs=pl.BlockSpec((BLK, C), lambda j, bs_ref: (j, 0)),
    )
    return pl.pallas_call(
        _experts_body,
        grid_spec=grid_spec,
        out_shape=jax.ShapeDtypeStruct((PAD, C), jnp.bfloat16),
    )(bs, x_padded, exp_w1, exp_b1.reshape(E, 1, C),
      exp_w2, exp_b2.reshape(E, 1, C))


def _epilogue_body(y_ref, dest_ref, p_ref, s_ref, out_ref):
    dest_row = dest_ref[...]
    jrow = jax.lax.broadcasted_iota(jnp.int32, (PAD, T), 0)
    perm = (jrow == dest_row).astype(jnp.bfloat16)
    y_tok = jax.lax.dot_general(perm, y_ref[...], (((0,), (0,)), ((), ())),
                                preferred_element_type=jnp.float32)
    z = y_tok * p_ref[...]
    z3 = z.reshape(B, HW, C)
    out_ref[...] = jnp.transpose(z3, (0, 2, 1)) + s_ref[...]


def _epilogue(y_padded, dest_row, p, s32r):
    return pl.pallas_call(
        _epilogue_body,
        out_shape=jax.ShapeDtypeStruct((B, C, HW), jnp.float32),
    )(y_padded, dest_row, p, s32r)


def kernel(s4, s8, s16, s32, gate_w1, gate_b1, gate_w2, gate_b2,
           exp_w1, exp_b1, exp_w2, exp_b2):
    s32r = s32.reshape(B, C, HW)

    idx2, p2, tok_f = _gate(s32r, gate_w1, gate_b1, gate_w2, gate_b2)
    idx_flat = idx2.reshape(T)
    x_padded, dest, bs = _dispatch(idx_flat, tok_f)
    y_padded = _grouped_experts(x_padded, bs, exp_w1, exp_b1, exp_w2, exp_b2)
    s32_out = _epilogue(y_padded, dest.reshape(1, T), p2,
                        s32r).reshape(B, C, 16, 16)

    return (s4, s8, s16, s32_out)


# speedup vs baseline: 1.4282x; 1.1281x over previous
import jax
import jax.numpy as jnp
from jax import lax
from jax.experimental import pallas as pl
from jax.experimental.pallas import tpu as pltpu
from jax.experimental.pallas import tpu_sc as plsc

B = 4
E = 8
C = 512
HW = 256
T = B * HW
EPAD = 128
BLK = 128
NB = T // BLK + E
PAD = NB * BLK
NC, NS, L = 2, 16, 16
NW = NC * NS
CH = T // NW
NV = T // L


def _gate_body(tok_ref, gw1_ref, gb1_ref, gw2_ref, gb2_ref,
               idx_ref, p_ref):
    tok = tok_ref[...]
    g1 = jax.nn.gelu(
        jax.lax.dot_general(tok, gw1_ref[...], (((1,), (0,)), ((), ())),
                            preferred_element_type=jnp.float32)
        + gb1_ref[...])
    gw2p = jnp.concatenate(
        [gw2_ref[...], jnp.zeros((C, EPAD - E), jnp.float32)], axis=1)
    logits = jax.lax.dot_general(g1, gw2p, (((1,), (0,)), ((), ())),
                                 preferred_element_type=jnp.float32)
    logits = logits + jnp.concatenate(
        [gb2_ref[...], jnp.zeros((1, EPAD - E), jnp.float32)], axis=1)
    col = jax.lax.broadcasted_iota(jnp.int32, (T, EPAD), 1)
    logits = jnp.where(col < E, logits, -1e30)
    m = jnp.max(logits, axis=1, keepdims=True)
    ex = jnp.exp(logits - m)
    denom = jnp.sum(ex, axis=1, keepdims=True)
    p_ref[...] = 1.0 / denom
    idx_ref[...] = jnp.min(jnp.where(logits == m, col, EPAD),
                           axis=1, keepdims=True).reshape(T // EPAD, EPAD)


def _gate(tok, gate_w1, gate_b1, gate_w2, gate_b2):
    return pl.pallas_call(
        _gate_body,
        out_shape=(jax.ShapeDtypeStruct((T // EPAD, EPAD), jnp.int32),
                   jax.ShapeDtypeStruct((T, 1), jnp.float32)),
    )(tok, gate_w1, gate_b1.reshape(1, C), gate_w2, gate_b2.reshape(1, E))


def _vgather(v, i):
    return lax.gather(
        v, i[:, None],
        lax.GatherDimensionNumbers(offset_dims=(), collapsed_slice_dims=(0,),
                                   start_index_map=(0,)),
        slice_sizes=(1,),
        mode=lax.GatherScatterMode.PROMISE_IN_BOUNDS)


def _worker_dispatch_math(read_vreg, wid):
    iota = lax.iota(jnp.int32, L)
    zero = jnp.zeros((L,), jnp.int32)

    def hist_step(k, carry):
        tot, pre = carry
        v = read_vreg(k)
        flag = jnp.where(k < 2 * wid, 1, 0)
        new_tot, new_pre = [], []
        for e in range(E):
            m = jnp.where(v == e, 1, 0)
            new_tot.append(tot[e] + m)
            new_pre.append(pre[e] + m * flag)
        return tuple(new_tot), tuple(new_pre)

    tot, pre = lax.fori_loop(0, NV, hist_step,
                             (tuple(zero for _ in range(E)),
                              tuple(zero for _ in range(E))))
    hv = zero
    pv = zero
    for e in range(E):
        lane = jnp.where(iota == e, 1, 0)
        hv = hv + lane * jnp.sum(tot[e])
        pv = pv + lane * jnp.sum(pre[e])

    nblk = (hv + (BLK - 1)) // BLK
    bs_incl = jnp.cumsum(nblk)
    padded_off = (bs_incl - nblk) * BLK
    base = padded_off + pv

    a = read_vreg(2 * wid)
    b = read_vreg(2 * wid + 1)
    cnt_a = zero
    intra_a = zero
    intra_b = zero
    for e in range(E):
        ma = jnp.where(a == e, 1, 0)
        mb = jnp.where(b == e, 1, 0)
        ca = jnp.cumsum(ma)
        cb = jnp.cumsum(mb)
        intra_a = jnp.where(a == e, ca - 1, intra_a)
        cnt_a = cnt_a + jnp.where(iota == e, 1, 0) * jnp.sum(ma)
        intra_b = jnp.where(b == e, cb - 1, intra_b)
    dest_a = _vgather(base, a) + intra_a
    dest_b = _vgather(base + cnt_a, b) + intra_b

    shifted = _vgather(bs_incl, jnp.maximum(iota - 1, 0))
    bsx = shifted * jnp.where(iota == 0, 0, 1)
    bsx = bsx * jnp.where(iota > E, 0, 1) + jnp.where(iota > E, NB, 0)
    return dest_a, dest_b, bsx, hv


def _dispatch_body(idx_hbm, tok_hbm, xpad_hbm, dest_hbm,
                   bs_hbm, idx_all, dest_v, rows_v, bs_v, sem):
    wid = lax.axis_index("s") * NC + lax.axis_index("c")
    pltpu.sync_copy(idx_hbm, idx_all)
    read = lambda k: idx_all[pl.ds(k * L, L)]
    dest_a, dest_b, bsx, hv = _worker_dispatch_math(read, wid)

    dest_v[pl.ds(0, L)] = dest_a
    dest_v[pl.ds(L, L)] = dest_b
    pltpu.sync_copy(dest_v, dest_hbm.at[pl.ds(wid * CH, CH)])

    pltpu.sync_copy(tok_hbm.at[pl.ds(wid * CH, CH)], rows_v)
    pltpu.async_copy(rows_v, xpad_hbm.at[dest_v], sem).wait()

    @pl.when(wid == 0)
    def _():
        bs_v[pl.ds(0, L)] = bsx
        bs_v[pl.ds(L, L)] = hv
        pltpu.sync_copy(bs_v, bs_hbm)


def _dispatch(idx, tok_bf):
    mesh = plsc.VectorSubcoreMesh(core_axis_name="c", subcore_axis_name="s",
                                  num_cores=NC, num_subcores=NS)
    return pl.kernel(
        _dispatch_body,
        out_type=(jax.ShapeDtypeStruct((PAD, C), jnp.float32),
                  jax.ShapeDtypeStruct((T,), jnp.int32),
                  jax.ShapeDtypeStruct((32,), jnp.int32)),
        mesh=mesh,
        scratch_types=[
            pltpu.VMEM((T,), jnp.int32),
            pltpu.VMEM((CH,), jnp.int32),
            pltpu.VMEM((CH, C), jnp.float32),
            pltpu.VMEM((2 * L,), jnp.int32),
            pltpu.SemaphoreType.DMA,
        ],
        compiler_params=pltpu.CompilerParams(needs_layout_passes=False),
    )(idx, tok_bf)


def _experts_body(bs_ref, x_ref, w1_ref, b1_ref, w2_ref, b2_ref, out_ref):
    j = pl.program_id(0)
    used = bs_ref[E]

    @pl.when(j < used)
    def _():
        e = _expert_of_block(j, bs_ref)
        h = jax.nn.gelu(
            jax.lax.dot_general(x_ref[...].astype(jnp.bfloat16),
                                w1_ref[0].astype(jnp.bfloat16),
                                (((1,), (0,)), ((), ())),
                                preferred_element_type=jnp.float32)
            + b1_ref[0])
        y = jax.lax.dot_general(
            h.astype(jnp.bfloat16), w2_ref[0].astype(jnp.bfloat16),
            (((1,), (0,)), ((), ())),
            preferred_element_type=jnp.float32) + b2_ref[0]
        row = jax.lax.broadcasted_iota(jnp.int32, (BLK, 1), 0)
        row_in_region = row + (j - _bs_at(bs_ref, e)) * BLK
        cnt = _cnt_at(bs_ref, e)
        out_ref[...] = jnp.where(row_in_region < cnt, y, 0.0).astype(
            jnp.bfloat16)

    @pl.when(j >= used)
    def _():
        out_ref[...] = jnp.zeros((BLK, C), jnp.bfloat16)


def _expert_of_block(j, bs_ref):
    e = jnp.int32(0)
    for k in range(1, E):
        e = e + jnp.where(j >= bs_ref[k], 1, 0).astype(jnp.int32)
    return e


def _bs_at(bs_ref, e):
    v = jnp.int32(0)
    for k in range(E):
        v = v + jnp.where(e == k, bs_ref[k], 0).astype(jnp.int32)
    return v


def _cnt_at(bs_ref, e):
    v = jnp.int32(0)
    for k in range(E):
        v = v + jnp.where(e == k, bs_ref[L + k], 0).astype(jnp.int32)
    return v


def _grouped_experts(x_padded, bs, exp_w1, exp_b1, exp_w2, exp_b2):
    grid_spec = pltpu.PrefetchScalarGridSpec(
        num_scalar_prefetch=1,
        grid=(NB,),
        in_specs=[
            pl.BlockSpec((BLK, C), lambda j, bs_ref: (j, 0)),
            pl.BlockSpec((1, C, C),
                         lambda j, bs_ref: (_expert_of_block(j, bs_ref), 0, 0)),
            pl.BlockSpec((1, 1, C),
                         lambda j, bs_ref: (_expert_of_block(j, bs_ref), 0, 0)),
            pl.BlockSpec((1, C, C),
                         lambda j, bs_ref: (_expert_of_block(j, bs_ref), 0, 0)),
            pl.BlockSpec((1, 1, C),
                         lambda j, bs_ref: (_expert_of_block(j, bs_ref), 0, 0)),
        ],
        out_specs=pl.BlockSpec((BLK, C), lambda j, bs_ref: (j, 0)),
    )
    return pl.pallas_call(
        _experts_body,
        grid_spec=grid_spec,
        out_shape=jax.ShapeDtypeStruct((PAD, C), jnp.bfloat16),
    )(bs, x_padded, exp_w1, exp_b1.reshape(E, 1, C),
      exp_w2, exp_b2.reshape(E, 1, C))


def _epilogue_body(y_ref, dest_ref, p_ref, tok_ref, out_ref):
    dest_row = dest_ref[...]
    jrow = jax.lax.broadcasted_iota(jnp.int32, (PAD, T), 0)
    perm = (jrow == dest_row).astype(jnp.bfloat16)
    y_tok = jax.lax.dot_general(perm, y_ref[...], (((0,), (0,)), ((), ())),
                                preferred_element_type=jnp.float32)
    out_ref[...] = y_tok * p_ref[...] + tok_ref[...]


def _epilogue(y_padded, dest_row, p, tok):
    return pl.pallas_call(
        _epilogue_body,
        out_shape=jax.ShapeDtypeStruct((T, C), jnp.float32),
    )(y_padded, dest_row, p, tok)


def kernel(s4, s8, s16, s32, gate_w1, gate_b1, gate_w2, gate_b2,
           exp_w1, exp_b1, exp_w2, exp_b2):
    tok = jnp.transpose(s32, (0, 2, 3, 1)).reshape(T, C)

    idx8, p2 = _gate(tok, gate_w1, gate_b1, gate_w2, gate_b2)
    x_padded, dest, bs = _dispatch(idx8.reshape(T), tok)
    y_padded = _grouped_experts(x_padded, bs, exp_w1, exp_b1, exp_w2, exp_b2)
    y_tok = _epilogue(y_padded, dest.reshape(1, T), p2, tok)
    s32_out = jnp.transpose(y_tok.reshape(B, 16, 16, C), (0, 3, 1, 2))

    return (s4, s8, s16, s32_out)


# speedup vs baseline: 1.5347x; 1.0746x over previous
import jax
import jax.numpy as jnp
from jax import lax
from jax.experimental import pallas as pl
from jax.experimental.pallas import tpu as pltpu
from jax.experimental.pallas import tpu_sc as plsc

B = 4
E = 8
C = 512
HW = 256
T = B * HW
EPAD = 128
BLK = 128
NB = T // BLK + E
PAD = NB * BLK
NC, NS, L = 2, 16, 16
NW = NC * NS
CH = T // NW
NV = T // L


def _gate_body(tok_ref, gw1_ref, gb1_ref, gw2_ref, gb2_ref,
               idx_ref, p_ref):
    tok = tok_ref[...]
    g1 = jax.nn.gelu(
        jax.lax.dot_general(tok, gw1_ref[...], (((1,), (0,)), ((), ())),
                            preferred_element_type=jnp.float32)
        + gb1_ref[...])
    gw2p = jnp.concatenate(
        [gw2_ref[...], jnp.zeros((C, EPAD - E), jnp.float32)], axis=1)
    logits = jax.lax.dot_general(g1, gw2p, (((1,), (0,)), ((), ())),
                                 preferred_element_type=jnp.float32)
    logits = logits + jnp.concatenate(
        [gb2_ref[...], jnp.zeros((1, EPAD - E), jnp.float32)], axis=1)
    col = jax.lax.broadcasted_iota(jnp.int32, (T, EPAD), 1)
    logits = jnp.where(col < E, logits, -1e30)
    m = jnp.max(logits, axis=1, keepdims=True)
    ex = jnp.exp(logits - m)
    denom = jnp.sum(ex, axis=1, keepdims=True)
    p_ref[...] = 1.0 / denom
    idx_ref[...] = jnp.min(jnp.where(logits == m, col, EPAD),
                           axis=1, keepdims=True).reshape(T // EPAD, EPAD)


def _gate(tok, gate_w1, gate_b1, gate_w2, gate_b2):
    return pl.pallas_call(
        _gate_body,
        out_shape=(jax.ShapeDtypeStruct((T // EPAD, EPAD), jnp.int32),
                   jax.ShapeDtypeStruct((T, 1), jnp.float32)),
    )(tok, gate_w1, gate_b1.reshape(1, C), gate_w2, gate_b2.reshape(1, E))


def _vgather(v, i):
    return lax.gather(
        v, i[:, None],
        lax.GatherDimensionNumbers(offset_dims=(), collapsed_slice_dims=(0,),
                                   start_index_map=(0,)),
        slice_sizes=(1,),
        mode=lax.GatherScatterMode.PROMISE_IN_BOUNDS)


def _worker_dispatch_math(read_vreg, wid):
    iota = lax.iota(jnp.int32, L)
    zero = jnp.zeros((L,), jnp.int32)

    def hist_step(k, carry):
        tot, pre = carry
        v = read_vreg(k)
        flag = jnp.where(k < 2 * wid, 1, 0)
        new_tot, new_pre = [], []
        for e in range(E):
            m = jnp.where(v == e, 1, 0)
            new_tot.append(tot[e] + m)
            new_pre.append(pre[e] + m * flag)
        return tuple(new_tot), tuple(new_pre)

    tot, pre = lax.fori_loop(0, NV, hist_step,
                             (tuple(zero for _ in range(E)),
                              tuple(zero for _ in range(E))))
    hv = zero
    pv = zero
    for e in range(E):
        lane = jnp.where(iota == e, 1, 0)
        hv = hv + lane * jnp.sum(tot[e])
        pv = pv + lane * jnp.sum(pre[e])

    nblk = (hv + (BLK - 1)) // BLK
    bs_incl = jnp.cumsum(nblk)
    padded_off = (bs_incl - nblk) * BLK
    base = padded_off + pv

    a = read_vreg(2 * wid)
    b = read_vreg(2 * wid + 1)
    cnt_a = zero
    intra_a = zero
    intra_b = zero
    for e in range(E):
        ma = jnp.where(a == e, 1, 0)
        mb = jnp.where(b == e, 1, 0)
        ca = jnp.cumsum(ma)
        cb = jnp.cumsum(mb)
        intra_a = jnp.where(a == e, ca - 1, intra_a)
        cnt_a = cnt_a + jnp.where(iota == e, 1, 0) * jnp.sum(ma)
        intra_b = jnp.where(b == e, cb - 1, intra_b)
    dest_a = _vgather(base, a) + intra_a
    dest_b = _vgather(base + cnt_a, b) + intra_b

    shifted = _vgather(bs_incl, jnp.maximum(iota - 1, 0))
    bsx = shifted * jnp.where(iota == 0, 0, 1)
    bsx = bsx * jnp.where(iota > E, 0, 1) + jnp.where(iota > E, NB, 0)
    return dest_a, dest_b, bsx, hv


def _dispatch_body(idx_hbm, tok_hbm, xpad_hbm, dest_hbm,
                   bs_hbm, idx_all, dest_v, rows_v, bs_v, sem):
    wid = lax.axis_index("s") * NC + lax.axis_index("c")
    pltpu.sync_copy(idx_hbm, idx_all)
    read = lambda k: idx_all[pl.ds(k * L, L)]
    dest_a, dest_b, bsx, hv = _worker_dispatch_math(read, wid)

    dest_v[pl.ds(0, L)] = dest_a
    dest_v[pl.ds(L, L)] = dest_b
    pltpu.sync_copy(dest_v, dest_hbm.at[pl.ds(wid * CH, CH)])

    pltpu.sync_copy(tok_hbm.at[pl.ds(wid * CH, CH)], rows_v)
    pltpu.async_copy(rows_v, xpad_hbm.at[dest_v], sem).wait()

    @pl.when(wid == 0)
    def _():
        bs_v[pl.ds(0, L)] = bsx
        bs_v[pl.ds(L, L)] = hv
        pltpu.sync_copy(bs_v, bs_hbm)


def _dispatch(idx, tok_bf):
    mesh = plsc.VectorSubcoreMesh(core_axis_name="c", subcore_axis_name="s",
                                  num_cores=NC, num_subcores=NS)
    return pl.kernel(
        _dispatch_body,
        out_type=(jax.ShapeDtypeStruct((PAD, C), jnp.float32),
                  jax.ShapeDtypeStruct((T,), jnp.int32),
                  jax.ShapeDtypeStruct((32,), jnp.int32)),
        mesh=mesh,
        scratch_types=[
            pltpu.VMEM((T,), jnp.int32),
            pltpu.VMEM((CH,), jnp.int32),
            pltpu.VMEM((CH, C), jnp.float32),
            pltpu.VMEM((2 * L,), jnp.int32),
            pltpu.SemaphoreType.DMA,
        ],
        compiler_params=pltpu.CompilerParams(needs_layout_passes=False),
    )(idx, tok_bf)


def _experts_body(bs_ref, x_ref, w1_ref, b1_ref, w2_ref, b2_ref, out_ref):
    j = pl.program_id(0)
    used = bs_ref[E]

    @pl.when(j < used)
    def _():
        e = _expert_of_block(j, bs_ref)
        h = jax.nn.gelu(
            jax.lax.dot_general(x_ref[...].astype(jnp.bfloat16),
                                w1_ref[0].astype(jnp.bfloat16),
                                (((1,), (0,)), ((), ())),
                                preferred_element_type=jnp.float32)
            + b1_ref[pl.ds(e, 1), :])
        y = jax.lax.dot_general(
            h.astype(jnp.bfloat16), w2_ref[0].astype(jnp.bfloat16),
            (((1,), (0,)), ((), ())),
            preferred_element_type=jnp.float32) + b2_ref[pl.ds(e, 1), :]
        row = jax.lax.broadcasted_iota(jnp.int32, (BLK, 1), 0)
        row_in_region = row + (j - _bs_at(bs_ref, e)) * BLK
        cnt = _cnt_at(bs_ref, e)
        out_ref[...] = jnp.where(row_in_region < cnt, y, 0.0).astype(
            jnp.bfloat16)

    @pl.when(j >= used)
    def _():
        out_ref[...] = jnp.zeros((BLK, C), jnp.bfloat16)


def _expert_of_block(j, bs_ref):
    e = jnp.int32(0)
    for k in range(1, E):
        e = e + jnp.where(j >= bs_ref[k], 1, 0).astype(jnp.int32)
    return e


def _bs_at(bs_ref, e):
    v = jnp.int32(0)
    for k in range(E):
        v = v + jnp.where(e == k, bs_ref[k], 0).astype(jnp.int32)
    return v


def _cnt_at(bs_ref, e):
    v = jnp.int32(0)
    for k in range(E):
        v = v + jnp.where(e == k, bs_ref[L + k], 0).astype(jnp.int32)
    return v


def _grouped_experts(x_padded, bs, exp_w1, exp_b1, exp_w2, exp_b2):
    grid_spec = pltpu.PrefetchScalarGridSpec(
        num_scalar_prefetch=1,
        grid=(NB,),
        in_specs=[
            pl.BlockSpec((BLK, C), lambda j, bs_ref: (j, 0)),
            pl.BlockSpec((1, C, C),
                         lambda j, bs_ref: (_expert_of_block(j, bs_ref), 0, 0)),
            pl.BlockSpec((E, C), lambda j, bs_ref: (0, 0)),
            pl.BlockSpec((1, C, C),
                         lambda j, bs_ref: (_expert_of_block(j, bs_ref), 0, 0)),
            pl.BlockSpec((E, C), lambda j, bs_ref: (0, 0)),
        ],
        out_specs=pl.BlockSpec((BLK, C), lambda j, bs_ref: (j, 0)),
    )
    return pl.pallas_call(
        _experts_body,
        grid_spec=grid_spec,
        out_shape=jax.ShapeDtypeStruct((PAD, C), jnp.bfloat16),
    )(bs, x_padded, exp_w1, exp_b1, exp_w2, exp_b2)


S4N = 4 * 64 * 128 * 128
S4CH = S4N // NW
S4STEP = 65536


def _s4copy_body(s4_hbm, bs_hbm, s4o_hbm, buf_v, sem):
    wid = lax.axis_index("s") * NC + lax.axis_index("c")
    base = wid * S4CH
    for k in range(S4CH // S4STEP):
        off = base + k * S4STEP
        pltpu.sync_copy(s4_hbm.at[pl.ds(off, S4STEP)], buf_v)
        pltpu.sync_copy(buf_v, s4o_hbm.at[pl.ds(off, S4STEP)])


def _s4copy(s4flat, bs):
    mesh = plsc.VectorSubcoreMesh(core_axis_name="c", subcore_axis_name="s",
                                  num_cores=NC, num_subcores=NS)
    return pl.kernel(
        _s4copy_body,
        out_type=jax.ShapeDtypeStruct((S4N,), jnp.float32),
        mesh=mesh,
        scratch_types=[
            pltpu.VMEM((S4STEP,), jnp.float32),
            pltpu.SemaphoreType.DMA,
        ],
        compiler_params=pltpu.CompilerParams(needs_layout_passes=False),
    )(s4flat, bs)


def _epilogue_body(y_ref, dest_ref, p_ref, tok_ref, out_ref):
    dest_row = dest_ref[...]
    jrow = jax.lax.broadcasted_iota(jnp.int32, (PAD, T), 0)
    perm = (jrow == dest_row).astype(jnp.bfloat16)
    y_tok = jax.lax.dot_general(perm, y_ref[...], (((0,), (0,)), ((), ())),
                                preferred_element_type=jnp.float32)
    out_ref[...] = y_tok * p_ref[...] + tok_ref[...]


def _epilogue(y_padded, dest_row, p, tok):
    return pl.pallas_call(
        _epilogue_body,
        out_shape=jax.ShapeDtypeStruct((T, C), jnp.float32),
    )(y_padded, dest_row, p, tok)


def kernel(s4, s8, s16, s32, gate_w1, gate_b1, gate_w2, gate_b2,
           exp_w1, exp_b1, exp_w2, exp_b2):
    tok = jnp.transpose(s32, (0, 2, 3, 1)).reshape(T, C)

    idx8, p2 = _gate(tok, gate_w1, gate_b1, gate_w2, gate_b2)
    x_padded, dest, bs = _dispatch(idx8.reshape(T), tok)
    y_padded = _grouped_experts(x_padded, bs, exp_w1, exp_b1, exp_w2, exp_b2)
    y_tok = _epilogue(y_padded, dest.reshape(1, T), p2, tok)
    s32_out = jnp.transpose(y_tok.reshape(B, 16, 16, C), (0, 3, 1, 2))
    s4_out = _s4copy(s4.reshape(S4N), bs).reshape(s4.shape)

    return (s4_out, s8, s16, s32_out)
